# Initial kernel scaffold; baseline (speedup 1.0000x reference)
import jax
import jax.numpy as jnp
from jax import lax
from jax.experimental import pallas as pl
from jax.experimental.pallas import tpu as pltpu
from jax.experimental.pallas import tpu_sc as plsc

_B = 2
_N = 10000
_E = 160000
_IN_F = 128
_OUT_F = 64
_ALPHA = 0.2

_LANES = 16
_NS = 16
_CHUNK = 128
_CPB = 10
_EB = _CHUNK * _CPB
_NBLK = _E // _EB
_FPT = _OUT_F // _NS
_NROW = _N // _LANES


def _prep_body(x_ref, W_ref, a_ref, hT_ref, ee_ref):
    W0 = W_ref[0]
    A = a_ref[0]
    for b in range(_B):
        hTb = lax.dot_general(W0, x_ref[b], (((0,), (1,)), ((), ())),
                              preferred_element_type=jnp.float32)
        hT_ref[b] = hTb
        ee_ref[b] = lax.dot_general(A, hTb, (((0,), (0,)), ((), ())),
                                    preferred_element_type=jnp.float32)


def _gat_body(hT_hbm, ee_hbm, eidx_hbm, out_hbm,
              ee0_v, ee1_v,
              ht0, ht1, ht2, ht3,
              hp0, hp1, hp2, hp3,
              i0_v, i1_v, f_v,
              rowsum_sh, f_sh):
    c = lax.axis_index("c")
    s = lax.axis_index("s")
    b = c
    hts = [ht0, ht1, ht2, ht3]
    hps = [hp0, hp1, hp2, hp3]

    pltpu.sync_copy(ee_hbm.at[b, 0], ee0_v)
    pltpu.sync_copy(ee_hbm.at[b, 1], ee1_v)
    for j in range(_FPT):
        pltpu.sync_copy(hT_hbm.at[b, s * _FPT + j], hts[j])

    @pl.loop(0, _NROW)
    def _zero(i):
        z = jnp.zeros((_LANES,), jnp.float32)
        for j in range(_FPT):
            hps[j][pl.ds(i * _LANES, _LANES)] = z

    @pl.when(s == 0)
    def _zero_rowsum():
        pltpu.sync_copy(hp0, rowsum_sh)

    plsc.subcore_barrier()

    @pl.loop(s, _NBLK, step=_NS)
    def _stage_b(blk):
        pltpu.sync_copy(eidx_hbm.at[0, blk], i0_v)
        pltpu.sync_copy(eidx_hbm.at[1, blk], i1_v)
        for ch in range(_CPB):
            for k in range(_CHUNK // _LANES):
                sl = pl.ds(k * _LANES, _LANES)
                i0 = i0_v[ch, sl]
                i1 = i1_v[ch, sl]
                e = plsc.load_gather(ee0_v, [i0]) + plsc.load_gather(ee1_v, [i1])
                lr = jnp.where(e > 0, e, _ALPHA * e)
                f_v[ch, sl] = jnp.exp(-jnp.clip(lr, -50.0, 50.0))
        for ch in range(_CPB):
            pltpu.sync_copy(f_v.at[ch], rowsum_sh.at[i0_v.at[ch]], add=True)
        pltpu.sync_copy(f_v, f_sh.at[blk])

    plsc.subcore_barrier()

    @pl.loop(0, _NBLK)
    def _stage_c(blk):
        pltpu.sync_copy(eidx_hbm.at[0, blk], i0_v)
        pltpu.sync_copy(eidx_hbm.at[1, blk], i1_v)
        pltpu.sync_copy(f_sh.at[blk], f_v)
        for ch in range(_CPB):
            for k in range(_CHUNK // _LANES):
                sl = pl.ds(k * _LANES, _LANES)
                i0 = i0_v[ch, sl]
                i1 = i1_v[ch, sl]
                fv = f_v[ch, sl]
                for j in range(_FPT):
                    vals = plsc.load_gather(hts[j], [i1]) * fv
                    plsc.addupdate_scatter(hps[j], [i0], vals)

    pltpu.sync_copy(rowsum_sh, ee0_v)

    @pl.loop(0, _NROW)
    def _stage_d(i):
        sl = pl.ds(i * _LANES, _LANES)
        denom = ee0_v[sl] + 1e-20
        for j in range(_FPT):
            v = hps[j][sl] / denom
            hps[j][sl] = jnp.where(v != v, 0.0, v)

    for j in range(_FPT):
        pltpu.sync_copy(hps[j], out_hbm.at[b, s * _FPT + j])


def kernel(x, edge_index, W, a):
    hT, ee = pl.pallas_call(
        _prep_body,
        out_shape=[
            jax.ShapeDtypeStruct((_B, _OUT_F, _N), jnp.float32),
            jax.ShapeDtypeStruct((_B, 2, _N), jnp.float32),
        ],
    )(x, W, a)

    eidx = edge_index.reshape(2, _NBLK, _CPB, _CHUNK)

    gat = pl.kernel(
        _gat_body,
        out_type=jax.ShapeDtypeStruct((_B, _OUT_F, _N), jnp.float32),
        mesh=plsc.VectorSubcoreMesh(core_axis_name="c", subcore_axis_name="s"),
        scratch_types=[
            pltpu.VMEM((_N,), jnp.float32),
            pltpu.VMEM((_N,), jnp.float32),
            pltpu.VMEM((_N,), jnp.float32),
            pltpu.VMEM((_N,), jnp.float32),
            pltpu.VMEM((_N,), jnp.float32),
            pltpu.VMEM((_N,), jnp.float32),
            pltpu.VMEM((_N,), jnp.float32),
            pltpu.VMEM((_N,), jnp.float32),
            pltpu.VMEM((_N,), jnp.float32),
            pltpu.VMEM((_N,), jnp.float32),
            pltpu.VMEM((_CPB, _CHUNK), jnp.int32),
            pltpu.VMEM((_CPB, _CHUNK), jnp.int32),
            pltpu.VMEM((_CPB, _CHUNK), jnp.float32),
            pltpu.VMEM_SHARED((_N,), jnp.float32),
            pltpu.VMEM_SHARED((_NBLK, _CPB, _CHUNK), jnp.float32),
        ],
    )
    hpT = gat(hT, ee, eidx)
    return hpT.transpose(0, 2, 1)


# speedup vs baseline: 27.7030x; 27.7030x over previous
import jax
import jax.numpy as jnp
from jax import lax
from jax.experimental import pallas as pl
from jax.experimental.pallas import tpu as pltpu
from jax.experimental.pallas import tpu_sc as plsc

_B = 2
_N = 10000
_E = 160000
_IN_F = 128
_OUT_F = 64
_ALPHA = 0.2

_LANES = 16
_NS = 16
_CHUNK = 128
_CPB = 10
_EB = _CHUNK * _CPB
_NBLK = _E // _EB
_FPT = _OUT_F // _NS
_NROW = _N // _LANES


def _prep_body(x_ref, W_ref, a_ref, hT_ref, ee_ref):
    W0 = W_ref[0]
    A = a_ref[0]
    for b in range(_B):
        hTb = lax.dot_general(W0, x_ref[b], (((0,), (1,)), ((), ())),
                              preferred_element_type=jnp.float32)
        hT_ref[b] = hTb
        ee_ref[b] = lax.dot_general(A, hTb, (((0,), (0,)), ((), ())),
                                    preferred_element_type=jnp.float32)


def _gat_body(hT_hbm, ee_hbm, eidx_hbm, out_hbm,
              ee0_v, ee1_v,
              ht0, ht1, ht2, ht3,
              hp0, hp1, hp2, hp3,
              i0_v, i1_v, f_v,
              rowsum_sh):
    c = lax.axis_index("c")
    s = lax.axis_index("s")
    b = c
    hts = [ht0, ht1, ht2, ht3]
    hps = [hp0, hp1, hp2, hp3]

    pltpu.sync_copy(ee_hbm.at[b, 0], ee0_v)
    pltpu.sync_copy(ee_hbm.at[b, 1], ee1_v)
    for j in range(_FPT):
        pltpu.sync_copy(hT_hbm.at[b, s * _FPT + j], hts[j])

    @pl.loop(0, _NROW)
    def _zero(i):
        z = jnp.zeros((_LANES,), jnp.float32)
        for j in range(_FPT):
            hps[j][pl.ds(i * _LANES, _LANES)] = z

    @pl.when(s == 0)
    def _zero_rowsum():
        pltpu.sync_copy(hp0, rowsum_sh)

    plsc.subcore_barrier()

    @pl.loop(s, _NBLK, step=_NS)
    def _stage_b(blk):
        pltpu.sync_copy(eidx_hbm.at[0, blk], i0_v)
        pltpu.sync_copy(eidx_hbm.at[1, blk], i1_v)
        for ch in range(_CPB):
            for k in range(_CHUNK // _LANES):
                sl = pl.ds(k * _LANES, _LANES)
                i0 = i0_v[ch, sl]
                i1 = i1_v[ch, sl]
                e = plsc.load_gather(ee0_v, [i0]) + plsc.load_gather(ee1_v, [i1])
                lr = jnp.where(e > 0, e, _ALPHA * e)
                f_v[ch, sl] = jnp.exp(-jnp.clip(lr, -50.0, 50.0))
        for ch in range(_CPB):
            pltpu.sync_copy(f_v.at[ch], rowsum_sh.at[i0_v.at[ch]], add=True)

    plsc.subcore_barrier()

    @pl.loop(0, _NBLK)
    def _stage_c(blk):
        pltpu.sync_copy(eidx_hbm.at[0, blk], i0_v)
        pltpu.sync_copy(eidx_hbm.at[1, blk], i1_v)
        for ch in range(_CPB):
            for k in range(_CHUNK // _LANES):
                sl = pl.ds(k * _LANES, _LANES)
                i0 = i0_v[ch, sl]
                i1 = i1_v[ch, sl]
                e = plsc.load_gather(ee0_v, [i0]) + plsc.load_gather(ee1_v, [i1])
                lr = jnp.where(e > 0, e, _ALPHA * e)
                fv = jnp.exp(-jnp.clip(lr, -50.0, 50.0))
                for j in range(_FPT):
                    vals = plsc.load_gather(hts[j], [i1]) * fv
                    plsc.addupdate_scatter(hps[j], [i0], vals)

    pltpu.sync_copy(rowsum_sh, ee0_v)

    @pl.loop(0, _NROW)
    def _stage_d(i):
        sl = pl.ds(i * _LANES, _LANES)
        denom = ee0_v[sl] + 1e-20
        for j in range(_FPT):
            v = hps[j][sl] / denom
            hps[j][sl] = jnp.where(v != v, 0.0, v)

    for j in range(_FPT):
        pltpu.sync_copy(hps[j], out_hbm.at[b, s * _FPT + j])


def kernel(x, edge_index, W, a):
    hT, ee = pl.pallas_call(
        _prep_body,
        out_shape=[
            jax.ShapeDtypeStruct((_B, _OUT_F, _N), jnp.float32),
            jax.ShapeDtypeStruct((_B, 2, _N), jnp.float32),
        ],
    )(x, W, a)

    eidx = edge_index.reshape(2, _NBLK, _CPB, _CHUNK)

    gat = pl.kernel(
        _gat_body,
        out_type=jax.ShapeDtypeStruct((_B, _OUT_F, _N), jnp.float32),
        mesh=plsc.VectorSubcoreMesh(core_axis_name="c", subcore_axis_name="s"),
        compiler_params=pltpu.CompilerParams(needs_layout_passes=False),
        scratch_types=[
            pltpu.VMEM((_N,), jnp.float32),
            pltpu.VMEM((_N,), jnp.float32),
            pltpu.VMEM((_N,), jnp.float32),
            pltpu.VMEM((_N,), jnp.float32),
            pltpu.VMEM((_N,), jnp.float32),
            pltpu.VMEM((_N,), jnp.float32),
            pltpu.VMEM((_N,), jnp.float32),
            pltpu.VMEM((_N,), jnp.float32),
            pltpu.VMEM((_N,), jnp.float32),
            pltpu.VMEM((_N,), jnp.float32),
            pltpu.VMEM((_CPB, _CHUNK), jnp.int32),
            pltpu.VMEM((_CPB, _CHUNK), jnp.int32),
            pltpu.VMEM((_CPB, _CHUNK), jnp.float32),
            pltpu.VMEM_SHARED((_N,), jnp.float32),
        ],
    )
    hpT = gat(hT, ee, eidx)
    return hpT.transpose(0, 2, 1)


# speedup vs baseline: 40.3929x; 1.4581x over previous
import jax
import jax.numpy as jnp
from jax import lax
from jax.experimental import pallas as pl
from jax.experimental.pallas import tpu as pltpu
from jax.experimental.pallas import tpu_sc as plsc

_B = 2
_N = 10000
_E = 160000
_IN_F = 128
_OUT_F = 64
_ALPHA = 0.2

_LANES = 16
_NS = 16
_CHUNK = 128
_CPB = 25
_EB = _CHUNK * _CPB
_NBLK = _E // _EB
_FPT = _OUT_F // _NS
_NROW = _N // _LANES


def _prep_body(x_ref, W_ref, a_ref, hT_ref, ee_ref):
    W0 = W_ref[0]
    A = a_ref[0]
    for b in range(_B):
        hTb = lax.dot_general(W0, x_ref[b], (((0,), (1,)), ((), ())),
                              preferred_element_type=jnp.float32)
        hT_ref[b] = hTb
        ee_ref[b] = lax.dot_general(A, hTb, (((0,), (0,)), ((), ())),
                                    preferred_element_type=jnp.float32)


def _gat_body(hT_hbm, ee_hbm, eidx_hbm, eidxf_hbm, out_hbm,
              ee0_v, ee1_v,
              ht0, ht1, ht2, ht3,
              hp0, hp1, hp2, hp3,
              i0_v, i1_v, i0c0, i1c0, i0c1, i1c1, f_v,
              sem0, sem1,
              rowsum_sh):
    c = lax.axis_index("c")
    s = lax.axis_index("s")
    b = c
    hts = [ht0, ht1, ht2, ht3]
    hps = [hp0, hp1, hp2, hp3]
    i0b = [i0c0, i0c1]
    i1b = [i1c0, i1c1]
    sems = [sem0, sem1]

    pltpu.sync_copy(ee_hbm.at[b, 0], ee0_v)
    pltpu.sync_copy(ee_hbm.at[b, 1], ee1_v)
    for j in range(_FPT):
        pltpu.sync_copy(hT_hbm.at[b, s * _FPT + j], hts[j])

    @pl.loop(0, _NROW)
    def _zero(i):
        z = jnp.zeros((_LANES,), jnp.float32)
        for j in range(_FPT):
            hps[j][pl.ds(i * _LANES, _LANES)] = z

    @pl.when(s == 0)
    def _zero_rowsum():
        pltpu.sync_copy(hp0, rowsum_sh)

    plsc.subcore_barrier()

    @pl.loop(s, _NBLK, step=_NS)
    def _stage_b(blk):
        pltpu.sync_copy(eidx_hbm.at[0, blk], i0_v)
        pltpu.sync_copy(eidx_hbm.at[1, blk], i1_v)
        for ch in range(_CPB):
            for k in range(_CHUNK // _LANES):
                sl = pl.ds(k * _LANES, _LANES)
                i0 = i0_v[ch, sl]
                i1 = i1_v[ch, sl]
                e = plsc.load_gather(ee0_v, [i0]) + plsc.load_gather(ee1_v, [i1])
                lr = jnp.where(e > 0, e, _ALPHA * e)
                f_v[ch, sl] = jnp.exp(-jnp.clip(lr, -50.0, 50.0))
        for ch in range(_CPB):
            pltpu.sync_copy(f_v.at[ch], rowsum_sh.at[i0_v.at[ch]], add=True)

    plsc.subcore_barrier()

    def _start(blk, p):
        pltpu.async_copy(eidxf_hbm.at[0, blk], i0b[p], sems[p])
        pltpu.async_copy(eidxf_hbm.at[1, blk], i1b[p], sems[p])

    def _wait(blk, p):
        pltpu.make_async_copy(eidxf_hbm.at[0, blk], i0b[p], sems[p]).wait()
        pltpu.make_async_copy(eidxf_hbm.at[1, blk], i1b[p], sems[p]).wait()

    _start(0, 0)

    @pl.loop(0, _NBLK // 2)
    def _stage_c(g):
        for p in range(2):
            blk = g * 2 + p

            @pl.when(blk + 1 < _NBLK)
            def _prefetch():
                _start(blk + 1, (p + 1) % 2)

            _wait(blk, p)

            @pl.loop(0, _CPB)
            def _row(ch):
                for k in range(_CHUNK // _LANES):
                    sl = pl.ds(ch * _CHUNK + k * _LANES, _LANES)
                    i0 = i0b[p][sl]
                    i1 = i1b[p][sl]
                    e = (plsc.load_gather(ee0_v, [i0])
                         + plsc.load_gather(ee1_v, [i1]))
                    lr = jnp.where(e > 0, e, _ALPHA * e)
                    fv = jnp.exp(-jnp.clip(lr, -50.0, 50.0))
                    for j in range(_FPT):
                        vals = plsc.load_gather(hts[j], [i1]) * fv
                        plsc.addupdate_scatter(hps[j], [i0], vals)

    pltpu.sync_copy(rowsum_sh, ee0_v)

    @pl.loop(0, _NROW)
    def _stage_d(i):
        sl = pl.ds(i * _LANES, _LANES)
        denom = ee0_v[sl] + 1e-20
        for j in range(_FPT):
            v = hps[j][sl] / denom
            hps[j][sl] = jnp.where(v != v, 0.0, v)

    for j in range(_FPT):
        pltpu.sync_copy(hps[j], out_hbm.at[b, s * _FPT + j])


def kernel(x, edge_index, W, a):
    hT, ee = pl.pallas_call(
        _prep_body,
        out_shape=[
            jax.ShapeDtypeStruct((_B, _OUT_F, _N), jnp.float32),
            jax.ShapeDtypeStruct((_B, 2, _N), jnp.float32),
        ],
    )(x, W, a)

    eidx = edge_index.reshape(2, _NBLK, _CPB, _CHUNK)
    eidxf = edge_index.reshape(2, _NBLK, _EB)

    gat = pl.kernel(
        _gat_body,
        out_type=jax.ShapeDtypeStruct((_B, _OUT_F, _N), jnp.float32),
        mesh=plsc.VectorSubcoreMesh(core_axis_name="c", subcore_axis_name="s"),
        compiler_params=pltpu.CompilerParams(needs_layout_passes=False),
        scratch_types=[
            pltpu.VMEM((_N,), jnp.float32),
            pltpu.VMEM((_N,), jnp.float32),
            pltpu.VMEM((_N,), jnp.float32),
            pltpu.VMEM((_N,), jnp.float32),
            pltpu.VMEM((_N,), jnp.float32),
            pltpu.VMEM((_N,), jnp.float32),
            pltpu.VMEM((_N,), jnp.float32),
            pltpu.VMEM((_N,), jnp.float32),
            pltpu.VMEM((_N,), jnp.float32),
            pltpu.VMEM((_N,), jnp.float32),
            pltpu.VMEM((_CPB, _CHUNK), jnp.int32),
            pltpu.VMEM((_CPB, _CHUNK), jnp.int32),
            pltpu.VMEM((_EB,), jnp.int32),
            pltpu.VMEM((_EB,), jnp.int32),
            pltpu.VMEM((_EB,), jnp.int32),
            pltpu.VMEM((_EB,), jnp.int32),
            pltpu.VMEM((_CPB, _CHUNK), jnp.float32),
            pltpu.SemaphoreType.DMA,
            pltpu.SemaphoreType.DMA,
            pltpu.VMEM_SHARED((_N,), jnp.float32),
        ],
    )
    hpT = gat(hT, ee, eidx, eidxf)
    return hpT.transpose(0, 2, 1)


# speedup vs baseline: 55.5339x; 1.3748x over previous
import jax
import jax.numpy as jnp
from jax import lax
from jax.experimental import pallas as pl
from jax.experimental.pallas import tpu as pltpu
from jax.experimental.pallas import tpu_sc as plsc

_B = 2
_N = 10000
_E = 160000
_IN_F = 128
_OUT_F = 64
_ALPHA = 0.2

_LANES = 16
_NS = 16
_CHUNK = 128
_CPB = 25
_EB = _CHUNK * _CPB
_NBLK = _E // _EB
_FPT = _OUT_F // _NS
_NROW = _N // _LANES


def _prep_body(x_ref, W_ref, a_ref, hT_ref, ee_ref):
    W0 = W_ref[0]
    A = a_ref[0]
    for b in range(_B):
        hTb = lax.dot_general(W0, x_ref[b], (((0,), (1,)), ((), ())),
                              preferred_element_type=jnp.float32)
        hT_ref[b] = hTb
        ee_ref[b] = lax.dot_general(A, hTb, (((0,), (0,)), ((), ())),
                                    preferred_element_type=jnp.float32)


def _gat_body(hT_hbm, ee_hbm, eidx_hbm, eidxf_hbm, out_hbm, f_hbm,
              ee0_v, ee1_v,
              ht0, ht1, ht2, ht3,
              hp0, hp1, hp2, hp3,
              i0_v, i0c0, i1c0, i0c1, i1c1, fc0, fc1, f_v,
              sem0, sem1,
              rowsum_sh):
    c = lax.axis_index("c")
    s = lax.axis_index("s")
    b = c
    hts = [ht0, ht1, ht2, ht3]
    hps = [hp0, hp1, hp2, hp3]
    i0b = [i0c0, i0c1]
    i1b = [i1c0, i1c1]
    fb = [fc0, fc1]
    sems = [sem0, sem1]

    pltpu.sync_copy(ee_hbm.at[b, 0], ee0_v)
    pltpu.sync_copy(ee_hbm.at[b, 1], ee1_v)
    for j in range(_FPT):
        pltpu.sync_copy(hT_hbm.at[b, s * _FPT + j], hts[j])

    @pl.loop(0, _NROW)
    def _zero(i):
        z = jnp.zeros((_LANES,), jnp.float32)
        for j in range(_FPT):
            hps[j][pl.ds(i * _LANES, _LANES)] = z

    @pl.when(s == 0)
    def _zero_rowsum():
        pltpu.sync_copy(hp0, rowsum_sh)

    plsc.subcore_barrier()

    @pl.loop(s, _NBLK, step=_NS)
    def _stage_b(blk):
        pltpu.sync_copy(eidx_hbm.at[0, blk], i0_v)
        pltpu.sync_copy(eidxf_hbm.at[1, blk], i1c0)
        for ch in range(_CPB):
            for k in range(_CHUNK // _LANES):
                sl = pl.ds(k * _LANES, _LANES)
                i0 = i0_v[ch, sl]
                i1 = i1c0[pl.ds(ch * _CHUNK + k * _LANES, _LANES)]
                e = plsc.load_gather(ee0_v, [i0]) + plsc.load_gather(ee1_v, [i1])
                lr = jnp.where(e > 0, e, _ALPHA * e)
                f_v[pl.ds(ch * _CHUNK + k * _LANES, _LANES)] = (
                    jnp.exp(-jnp.clip(lr, -50.0, 50.0)))
        pltpu.sync_copy(f_v, f_hbm.at[b, blk])
        for ch in range(_CPB):
            pltpu.sync_copy(f_v.at[pl.ds(ch * _CHUNK, _CHUNK)],
                            rowsum_sh.at[i0_v.at[ch]], add=True)

    plsc.subcore_barrier()

    def _start(blk, p):
        pltpu.async_copy(eidxf_hbm.at[0, blk], i0b[p], sems[p])
        pltpu.async_copy(eidxf_hbm.at[1, blk], i1b[p], sems[p])
        pltpu.async_copy(f_hbm.at[b, blk], fb[p], sems[p])

    def _wait(blk, p):
        pltpu.make_async_copy(eidxf_hbm.at[0, blk], i0b[p], sems[p]).wait()
        pltpu.make_async_copy(eidxf_hbm.at[1, blk], i1b[p], sems[p]).wait()
        pltpu.make_async_copy(f_hbm.at[b, blk], fb[p], sems[p]).wait()

    _start(0, 0)

    @pl.loop(0, _NBLK // 2)
    def _stage_c(g):
        for p in range(2):
            blk = g * 2 + p

            @pl.when(blk + 1 < _NBLK)
            def _prefetch():
                _start(blk + 1, (p + 1) % 2)

            _wait(blk, p)

            @pl.loop(0, _CPB)
            def _row(ch):
                for k in range(_CHUNK // _LANES):
                    sl = pl.ds(ch * _CHUNK + k * _LANES, _LANES)
                    i0 = i0b[p][sl]
                    i1 = i1b[p][sl]
                    fv = fb[p][sl]
                    for j in range(_FPT):
                        vals = plsc.load_gather(hts[j], [i1]) * fv
                        plsc.addupdate_scatter(hps[j], [i0], vals)

    pltpu.sync_copy(rowsum_sh, ee0_v)

    @pl.loop(0, _NROW)
    def _stage_d(i):
        sl = pl.ds(i * _LANES, _LANES)
        denom = ee0_v[sl] + 1e-20
        for j in range(_FPT):
            v = hps[j][sl] / denom
            hps[j][sl] = jnp.where(v != v, 0.0, v)

    for j in range(_FPT):
        pltpu.sync_copy(hps[j], out_hbm.at[b, s * _FPT + j])


def kernel(x, edge_index, W, a):
    hT, ee = pl.pallas_call(
        _prep_body,
        out_shape=[
            jax.ShapeDtypeStruct((_B, _OUT_F, _N), jnp.float32),
            jax.ShapeDtypeStruct((_B, 2, _N), jnp.float32),
        ],
    )(x, W, a)

    eidx = edge_index.reshape(2, _NBLK, _CPB, _CHUNK)
    eidxf = edge_index.reshape(2, _NBLK, _EB)

    gat = pl.kernel(
        _gat_body,
        out_type=[
            jax.ShapeDtypeStruct((_B, _OUT_F, _N), jnp.float32),
            jax.ShapeDtypeStruct((_B, _NBLK, _EB), jnp.float32),
        ],
        mesh=plsc.VectorSubcoreMesh(core_axis_name="c", subcore_axis_name="s"),
        compiler_params=pltpu.CompilerParams(needs_layout_passes=False),
        scratch_types=[
            pltpu.VMEM((_N,), jnp.float32),
            pltpu.VMEM((_N,), jnp.float32),
            pltpu.VMEM((_N,), jnp.float32),
            pltpu.VMEM((_N,), jnp.float32),
            pltpu.VMEM((_N,), jnp.float32),
            pltpu.VMEM((_N,), jnp.float32),
            pltpu.VMEM((_N,), jnp.float32),
            pltpu.VMEM((_N,), jnp.float32),
            pltpu.VMEM((_N,), jnp.float32),
            pltpu.VMEM((_N,), jnp.float32),
            pltpu.VMEM((_CPB, _CHUNK), jnp.int32),
            pltpu.VMEM((_EB,), jnp.int32),
            pltpu.VMEM((_EB,), jnp.int32),
            pltpu.VMEM((_EB,), jnp.int32),
            pltpu.VMEM((_EB,), jnp.int32),
            pltpu.VMEM((_EB,), jnp.float32),
            pltpu.VMEM((_EB,), jnp.float32),
            pltpu.VMEM((_EB,), jnp.float32),
            pltpu.SemaphoreType.DMA,
            pltpu.SemaphoreType.DMA,
            pltpu.VMEM_SHARED((_N,), jnp.float32),
        ],
    )
    hpT, _ = gat(hT, ee, eidx, eidxf)
    return hpT.transpose(0, 2, 1)


# speedup vs baseline: 94.4211x; 1.7002x over previous
import jax
import jax.numpy as jnp
from jax import lax
from jax.experimental import pallas as pl
from jax.experimental.pallas import tpu as pltpu
from jax.experimental.pallas import tpu_sc as plsc

_B = 2
_N = 10000
_E = 160000
_IN_F = 128
_OUT_F = 64
_ALPHA = 0.2

_LANES = 16
_NS = 16
_CHUNK = 128
_CPB = 25
_EB = _CHUNK * _CPB
_NBLK = _E // _EB
_FPT = _OUT_F // _NS
_NROW = _N // _LANES


def _prep_body(x_ref, W_ref, a_ref, hT_ref, ee_ref):
    W0 = W_ref[0]
    A = a_ref[0]
    for b in range(_B):
        hTb = lax.dot_general(W0, x_ref[b], (((0,), (1,)), ((), ())),
                              preferred_element_type=jnp.float32)
        hT_ref[b] = hTb
        ee_ref[b] = lax.dot_general(A, hTb, (((0,), (0,)), ((), ())),
                                    preferred_element_type=jnp.float32)


def _gat_body(hT_hbm, ee_hbm, eidx_hbm, eidxf_hbm, out_hbm, f_hbm,
              ee0_v, ee1_v,
              ht0, ht1, ht2, ht3,
              hp0, hp1, hp2, hp3,
              i0_v, i0c0, i1c0, i0c1, i1c1, fc0, fc1, f_v,
              sem0, sem1,
              rowsum_sh):
    c = lax.axis_index("c")
    s = lax.axis_index("s")
    b = c
    hts = [ht0, ht1, ht2, ht3]
    hps = [hp0, hp1, hp2, hp3]
    i0b = [i0c0, i0c1]
    i1b = [i1c0, i1c1]
    fb = [fc0, fc1]
    sems = [sem0, sem1]

    pltpu.sync_copy(ee_hbm.at[b, 0], ee0_v)
    pltpu.sync_copy(ee_hbm.at[b, 1], ee1_v)
    for j in range(_FPT):
        pltpu.sync_copy(hT_hbm.at[b, s * _FPT + j], hts[j])

    @pl.loop(0, _NROW)
    def _zero(i):
        z = jnp.zeros((_LANES,), jnp.float32)
        for j in range(_FPT):
            hps[j][pl.ds(i * _LANES, _LANES)] = z

    @pl.when(s == 0)
    def _zero_rowsum():
        pltpu.sync_copy(hp0, rowsum_sh)

    plsc.subcore_barrier()

    @pl.loop(s, _NBLK, step=_NS)
    def _stage_b(blk):
        pltpu.sync_copy(eidx_hbm.at[0, blk], i0_v)
        pltpu.sync_copy(eidxf_hbm.at[1, blk], i1c0)
        for ch in range(_CPB):
            for k in range(_CHUNK // _LANES):
                sl = pl.ds(k * _LANES, _LANES)
                i0 = i0_v[ch, sl]
                i1 = i1c0[pl.ds(ch * _CHUNK + k * _LANES, _LANES)]
                e = plsc.load_gather(ee0_v, [i0]) + plsc.load_gather(ee1_v, [i1])
                lr = jnp.where(e > 0, e, _ALPHA * e)
                f_v[pl.ds(ch * _CHUNK + k * _LANES, _LANES)] = (
                    jnp.exp(-jnp.clip(lr, -50.0, 50.0)))
        pltpu.sync_copy(f_v, f_hbm.at[b, blk])
        for ch in range(_CPB):
            pltpu.sync_copy(f_v.at[pl.ds(ch * _CHUNK, _CHUNK)],
                            rowsum_sh.at[i0_v.at[ch]], add=True)

    plsc.subcore_barrier()

    def _start(blk, p):
        pltpu.async_copy(eidxf_hbm.at[0, blk], i0b[p], sems[p])
        pltpu.async_copy(eidxf_hbm.at[1, blk], i1b[p], sems[p])
        pltpu.async_copy(f_hbm.at[b, blk], fb[p], sems[p])

    def _wait(blk, p):
        pltpu.make_async_copy(eidxf_hbm.at[0, blk], i0b[p], sems[p]).wait()
        pltpu.make_async_copy(eidxf_hbm.at[1, blk], i1b[p], sems[p]).wait()
        pltpu.make_async_copy(f_hbm.at[b, blk], fb[p], sems[p]).wait()

    _start(0, 0)

    @pl.loop(0, _NBLK // 2)
    def _stage_c(g):
        for p in range(2):
            blk = g * 2 + p

            @pl.when(blk + 1 < _NBLK)
            def _prefetch():
                _start(blk + 1, (p + 1) % 2)

            _wait(blk, p)

            @plsc.parallel_loop(0, _CPB, step=1, unroll=2)
            def _row(ch):
                for k in range(_CHUNK // _LANES):
                    sl = pl.ds(ch * _CHUNK + k * _LANES, _LANES)
                    i0 = i0b[p][sl]
                    i1 = i1b[p][sl]
                    fv = fb[p][sl]
                    for j in range(_FPT):
                        vals = plsc.load_gather(hts[j], [i1]) * fv
                        plsc.addupdate_scatter(hps[j], [i0], vals)

    pltpu.sync_copy(rowsum_sh, ee0_v)

    @pl.loop(0, _NROW)
    def _stage_d(i):
        sl = pl.ds(i * _LANES, _LANES)
        denom = ee0_v[sl] + 1e-20
        for j in range(_FPT):
            v = hps[j][sl] / denom
            hps[j][sl] = jnp.where(v != v, 0.0, v)

    for j in range(_FPT):
        pltpu.sync_copy(hps[j], out_hbm.at[b, s * _FPT + j])


def kernel(x, edge_index, W, a):
    hT, ee = pl.pallas_call(
        _prep_body,
        out_shape=[
            jax.ShapeDtypeStruct((_B, _OUT_F, _N), jnp.float32),
            jax.ShapeDtypeStruct((_B, 2, _N), jnp.float32),
        ],
    )(x, W, a)

    eidx = edge_index.reshape(2, _NBLK, _CPB, _CHUNK)
    eidxf = edge_index.reshape(2, _NBLK, _EB)

    gat = pl.kernel(
        _gat_body,
        out_type=[
            jax.ShapeDtypeStruct((_B, _OUT_F, _N), jnp.float32),
            jax.ShapeDtypeStruct((_B, _NBLK, _EB), jnp.float32),
        ],
        mesh=plsc.VectorSubcoreMesh(core_axis_name="c", subcore_axis_name="s"),
        compiler_params=pltpu.CompilerParams(needs_layout_passes=False),
        scratch_types=[
            pltpu.VMEM((_N,), jnp.float32),
            pltpu.VMEM((_N,), jnp.float32),
            pltpu.VMEM((_N,), jnp.float32),
            pltpu.VMEM((_N,), jnp.float32),
            pltpu.VMEM((_N,), jnp.float32),
            pltpu.VMEM((_N,), jnp.float32),
            pltpu.VMEM((_N,), jnp.float32),
            pltpu.VMEM((_N,), jnp.float32),
            pltpu.VMEM((_N,), jnp.float32),
            pltpu.VMEM((_N,), jnp.float32),
            pltpu.VMEM((_CPB, _CHUNK), jnp.int32),
            pltpu.VMEM((_EB,), jnp.int32),
            pltpu.VMEM((_EB,), jnp.int32),
            pltpu.VMEM((_EB,), jnp.int32),
            pltpu.VMEM((_EB,), jnp.int32),
            pltpu.VMEM((_EB,), jnp.float32),
            pltpu.VMEM((_EB,), jnp.float32),
            pltpu.VMEM((_EB,), jnp.float32),
            pltpu.SemaphoreType.DMA,
            pltpu.SemaphoreType.DMA,
            pltpu.VMEM_SHARED((_N,), jnp.float32),
        ],
    )
    hpT, _ = gat(hT, ee, eidx, eidxf)
    return hpT.transpose(0, 2, 1)
